# Initial kernel scaffold; baseline (speedup 1.0000x reference)
import jax
import jax.numpy as jnp
from jax.experimental import pallas as pl


def kernel(x_pf, W1, b1, W2, b2, Wc, bc, Wo1, bo1, Wo2, bo2, Wo3, bo3, Wo4, bo4, batch_pf):
    raise NotImplementedError("write your pallas kernel here")


# speedup vs baseline: 9.0652x; 9.0652x over previous
import functools

import jax
import jax.numpy as jnp
from jax import lax
from jax.experimental import pallas as pl
from jax.experimental.pallas import tpu as pltpu

N = 8192
G = 128
H = 128
K = 24
RB = 128
NB = N // RB

_dg = functools.partial(lax.dot_general, precision=lax.Precision.HIGHEST,
                        preferred_element_type=jnp.float32)
_dgd = functools.partial(lax.dot_general, precision=lax.Precision.DEFAULT,
                         preferred_element_type=jnp.float32)
_STD = (((1,), (0,)), ((), ()))
_TRN = (((1,), (1,)), ((), ()))


def _elu(x):
    return jnp.where(x > 0, x, jnp.exp(jnp.minimum(x, 0.0)) - 1.0)


def _mlp_in_kernel(x_ref, w1_ref, b1_ref, w2_ref, b2_ref, o_ref):
    h1 = _elu(_dgd(x_ref[...], w1_ref[...], _STD) + b1_ref[...])
    o_ref[...] = _elu(_dgd(h1, w2_ref[...], _STD) + b2_ref[...])


def _edge_kernel(blo_ref, bhi_ref, rowlo_ref, rowhi_ref, h_ref,
                 wct_ref, wcb_ref, bc_ref, o_ref, dist_ref, pairs_ref):
    rb = pl.program_id(0)
    blo = blo_ref[rb]
    bhi = bhi_ref[rb]
    h_i = h_ref[pl.ds(rb * RB, RB), :]
    row_lo = rowlo_ref[0]
    row_hi = rowhi_ref[0]
    ones_row = jnp.ones((1, H), jnp.float32)

    def fill(cb, carry):
        hj = h_ref[pl.ds(cb * RB, RB), :]
        gmat = _dgd(h_i, hj, _TRN)
        n2j = _dg(ones_row, hj * hj, _TRN)
        d = n2j - 2.0 * gmat
        jglob = cb * RB + lax.broadcasted_iota(jnp.int32, (1, RB), 1)
        valid = (jglob >= row_lo) & (jglob < row_hi)
        dist_ref[cb] = jnp.where(valid, d, jnp.inf)
        return carry

    lax.fori_loop(blo, bhi, fill, 0)

    def kstep(k, m_prev):
        def mstep(cb, cur):
            d = dist_ref[cb]
            d = jnp.where(d > m_prev, d, jnp.inf)
            return jnp.minimum(cur, jnp.min(d, axis=1, keepdims=True))

        m_k = lax.fori_loop(blo, bhi, mstep, jnp.full((RB, 1), jnp.inf, jnp.float32))

        def gstep(cb, g_acc):
            oh = (dist_ref[cb] == m_k).astype(jnp.float32)
            hb = h_ref[pl.ds(cb * RB, RB), :]
            return g_acc + _dg(oh, hb, _STD)

        hrow_k = lax.fori_loop(blo, bhi, gstep, jnp.zeros((RB, H), jnp.float32))
        pairs_ref[pl.ds(k * RB, RB), :] = jnp.where(
            m_k < jnp.inf, hrow_k - h_i, 0.0)
        return m_k

    lax.fori_loop(0, K, kstep, jnp.full((RB, 1), -jnp.inf, jnp.float32))

    a_i = _dgd(h_i, wct_ref[...], _STD)
    p = _dgd(pairs_ref[...], wcb_ref[...], _STD)
    maxp = p[0:RB, :]
    for k in range(1, K):
        maxp = jnp.maximum(maxp, p[k * RB:(k + 1) * RB, :])
    o_ref[...] = _elu(a_i + bc_ref[...] + maxp)


def _pool_kernel(batchc_ref, h_ref, wo1_ref, bo1_ref, wo2_ref, bo2_ref,
                 wo3_ref, bo3_ref, wo4_ref, bo4_ref, o_ref):
    g_iota = lax.broadcasted_iota(jnp.int32, (G, 1), 0)

    def body(cb, acc):
        bj = batchc_ref[cb]
        oh = (bj == g_iota).astype(jnp.float32)
        hb = h_ref[pl.ds(cb * RB, RB), :]
        return acc + _dg(oh, hb, (((1,), (0,)), ((), ())))

    pooled = lax.fori_loop(0, NB, body, jnp.zeros((G, H), jnp.float32))
    o = _elu(_dg(pooled, wo1_ref[...], (((1,), (0,)), ((), ()))) + bo1_ref[...])
    o = _elu(_dg(o, wo2_ref[...], (((1,), (0,)), ((), ()))) + bo2_ref[...])
    o = _elu(_dg(o, wo3_ref[...], (((1,), (0,)), ((), ()))) + bo3_ref[...])
    o_ref[...] = _dg(o, wo4_ref[...], (((1,), (0,)), ((), ()))) + bo4_ref[...]


def _full2(shape):
    return pl.BlockSpec(shape, lambda i: (0,) * len(shape))


def kernel(x_pf, W1, b1, W2, b2, Wc, bc, Wo1, bo1, Wo2, bo2, Wo3, bo3, Wo4,
           bo4, batch_pf):
    f32 = jnp.float32
    batch = batch_pf.astype(jnp.int32)
    g_ids = jnp.arange(G, dtype=jnp.int32)
    seg_start = jnp.searchsorted(batch, g_ids, side='left').astype(jnp.int32)
    seg_end = jnp.searchsorted(batch, g_ids, side='right').astype(jnp.int32)
    row_lo = seg_start[batch]
    row_hi = seg_end[batch]
    rowlo3 = row_lo.reshape(NB, RB, 1)
    rowhi3 = row_hi.reshape(NB, RB, 1)
    blo = (seg_start[batch[0::RB]] // RB).astype(jnp.int32)
    bhi = ((seg_end[batch[RB - 1::RB]] + RB - 1) // RB).astype(jnp.int32)
    batchc = batch.reshape(NB, 1, RB)

    x16 = jnp.pad(x_pf, ((0, 0), (0, 1)))
    w1p = jnp.pad(W1, ((0, 1), (0, 0)))
    b1r = b1.reshape(1, -1)
    b2r = b2.reshape(1, -1)
    bcr = bc.reshape(1, -1)
    wct = Wc[:H]
    wcb = Wc[H:]

    h = pl.pallas_call(
        _mlp_in_kernel,
        grid=(NB,),
        in_specs=[
            pl.BlockSpec((RB, 16), lambda i: (i, 0)),
            _full2((16, H)), _full2((1, H)), _full2((H, H)), _full2((1, H)),
        ],
        out_specs=pl.BlockSpec((RB, H), lambda i: (i, 0)),
        out_shape=jax.ShapeDtypeStruct((N, H), f32),
        compiler_params=pltpu.CompilerParams(
            dimension_semantics=("arbitrary",)),
    )(x16, w1p, b1r, W2, b2r)

    for _ in range(3):
        h = pl.pallas_call(
            _edge_kernel,
            grid=(NB,),
            in_specs=[
                pl.BlockSpec(memory_space=pltpu.SMEM),
                pl.BlockSpec(memory_space=pltpu.SMEM),
                pl.BlockSpec((1, RB, 1), lambda i: (i, 0, 0)),
                pl.BlockSpec((1, RB, 1), lambda i: (i, 0, 0)),
                _full2((N, H)),
                _full2((H, H)),
                _full2((H, H)),
                _full2((1, H)),
            ],
            out_specs=pl.BlockSpec((RB, H), lambda i: (i, 0)),
            out_shape=jax.ShapeDtypeStruct((N, H), f32),
            scratch_shapes=[pltpu.VMEM((NB, RB, RB), f32),
                            pltpu.VMEM((K * RB, H), f32)],
            compiler_params=pltpu.CompilerParams(
                dimension_semantics=("arbitrary",)),
        )(blo, bhi, rowlo3, rowhi3, h, wct, wcb, bcr)

    out = pl.pallas_call(
        _pool_kernel,
        grid=(1,),
        in_specs=[
            pl.BlockSpec((NB, 1, RB), lambda i: (0, 0, 0)),
            _full2((N, H)),
            _full2((H, 64)), _full2((1, 64)),
            _full2((64, 32)), _full2((1, 32)),
            _full2((32, 32)), _full2((1, 32)),
            _full2((32, 8)), _full2((1, 8)),
        ],
        out_specs=pl.BlockSpec((G, 8), lambda i: (0, 0)),
        out_shape=jax.ShapeDtypeStruct((G, 8), f32),
        compiler_params=pltpu.CompilerParams(
            dimension_semantics=("arbitrary",)),
    )(batchc, h, Wo1, bo1.reshape(1, -1), Wo2, bo2.reshape(1, -1),
      Wo3, bo3.reshape(1, -1), Wo4, bo4.reshape(1, -1))

    return (out, batch_pf)


# speedup vs baseline: 15.8409x; 1.7474x over previous
import functools

import jax
import jax.numpy as jnp
from jax import lax
from jax.experimental import pallas as pl
from jax.experimental.pallas import tpu as pltpu

N = 8192
G = 128
H = 128
K = 24
RB = 128
NB = N // RB

_dg = functools.partial(lax.dot_general, precision=lax.Precision.HIGHEST,
                        preferred_element_type=jnp.float32)
_dgd = functools.partial(lax.dot_general, precision=lax.Precision.DEFAULT,
                         preferred_element_type=jnp.float32)
_STD = (((1,), (0,)), ((), ()))
_TRN = (((1,), (1,)), ((), ()))


def _elu(x):
    return jnp.where(x > 0, x, jnp.exp(jnp.minimum(x, 0.0)) - 1.0)


def _mlp_in_kernel(x_ref, w1_ref, b1_ref, w2_ref, b2_ref, o_ref):
    h1 = _elu(_dgd(x_ref[...], w1_ref[...], _STD) + b1_ref[...])
    o_ref[...] = _elu(_dgd(h1, w2_ref[...], _STD) + b2_ref[...])


def _edge_kernel(blo_ref, bhi_ref, rowlo_ref, rowhi_ref, h_ref,
                 wct_ref, wcb_ref, bc_ref, o_ref, dist_ref, mstack_ref,
                 gath_ref):
    rb = pl.program_id(0)
    blo = blo_ref[rb]
    bhi = bhi_ref[rb]
    h_i = h_ref[pl.ds(rb * RB, RB), :]
    row_lo = rowlo_ref[0]
    row_hi = rowhi_ref[0]
    ones_row = jnp.ones((1, H), jnp.float32)

    def fill(cb, carry):
        hj = h_ref[pl.ds(cb * RB, RB), :]
        gmat = _dgd(h_i, hj, _TRN)
        n2j = _dg(ones_row, hj * hj, _TRN)
        d = n2j - 2.0 * gmat
        jglob = cb * RB + lax.broadcasted_iota(jnp.int32, (1, RB), 1)
        valid = (jglob >= row_lo) & (jglob < row_hi)
        dist_ref[cb] = jnp.where(valid, d, jnp.inf)
        return carry

    lax.fori_loop(blo, bhi, fill, 0)

    def kstep(k, m_prev):
        def mstep(cb, cur):
            d = dist_ref[cb]
            d = jnp.where(d > m_prev, d, jnp.inf)
            return jnp.minimum(cur, jnp.min(d, axis=1, keepdims=True))

        m_k = lax.fori_loop(blo, bhi, mstep, jnp.full((RB, 1), jnp.inf, jnp.float32))
        mstack_ref[k] = m_k
        return m_k

    lax.fori_loop(0, K, kstep, jnp.full((RB, 1), -jnp.inf, jnp.float32))

    gath_ref[...] = jnp.zeros((K * RB, H), jnp.float32)

    def gstep(cb, carry):
        d = dist_ref[cb]
        ohs = jnp.concatenate(
            [(d == mstack_ref[kk]).astype(jnp.float32) for kk in range(K)],
            axis=0)
        hb = h_ref[pl.ds(cb * RB, RB), :]
        gath_ref[...] = gath_ref[...] + _dg(ohs, hb, _STD)
        return carry

    lax.fori_loop(blo, bhi, gstep, 0)

    pairs = jnp.concatenate(
        [jnp.where(mstack_ref[kk] < jnp.inf,
                   gath_ref[kk * RB:(kk + 1) * RB, :] - h_i, 0.0)
         for kk in range(K)], axis=0)
    a_i = _dgd(h_i, wct_ref[...], _STD)
    p = _dgd(pairs, wcb_ref[...], _STD)
    maxp = p[0:RB, :]
    for k in range(1, K):
        maxp = jnp.maximum(maxp, p[k * RB:(k + 1) * RB, :])
    o_ref[...] = _elu(a_i + bc_ref[...] + maxp)


def _pool_kernel(batchc_ref, h_ref, wo1_ref, bo1_ref, wo2_ref, bo2_ref,
                 wo3_ref, bo3_ref, wo4_ref, bo4_ref, o_ref):
    g_iota = lax.broadcasted_iota(jnp.int32, (G, 1), 0)

    def body(cb, acc):
        bj = batchc_ref[cb]
        oh = (bj == g_iota).astype(jnp.float32)
        hb = h_ref[pl.ds(cb * RB, RB), :]
        return acc + _dg(oh, hb, (((1,), (0,)), ((), ())))

    pooled = lax.fori_loop(0, NB, body, jnp.zeros((G, H), jnp.float32))
    o = _elu(_dg(pooled, wo1_ref[...], (((1,), (0,)), ((), ()))) + bo1_ref[...])
    o = _elu(_dg(o, wo2_ref[...], (((1,), (0,)), ((), ()))) + bo2_ref[...])
    o = _elu(_dg(o, wo3_ref[...], (((1,), (0,)), ((), ()))) + bo3_ref[...])
    o_ref[...] = _dg(o, wo4_ref[...], (((1,), (0,)), ((), ()))) + bo4_ref[...]


def _full2(shape):
    return pl.BlockSpec(shape, lambda i: (0,) * len(shape))


def kernel(x_pf, W1, b1, W2, b2, Wc, bc, Wo1, bo1, Wo2, bo2, Wo3, bo3, Wo4,
           bo4, batch_pf):
    f32 = jnp.float32
    batch = batch_pf.astype(jnp.int32)
    g_ids = jnp.arange(G, dtype=jnp.int32)
    seg_start = jnp.searchsorted(batch, g_ids, side='left').astype(jnp.int32)
    seg_end = jnp.searchsorted(batch, g_ids, side='right').astype(jnp.int32)
    row_lo = seg_start[batch]
    row_hi = seg_end[batch]
    rowlo3 = row_lo.reshape(NB, RB, 1)
    rowhi3 = row_hi.reshape(NB, RB, 1)
    blo = (seg_start[batch[0::RB]] // RB).astype(jnp.int32)
    bhi = ((seg_end[batch[RB - 1::RB]] + RB - 1) // RB).astype(jnp.int32)
    batchc = batch.reshape(NB, 1, RB)

    x16 = jnp.pad(x_pf, ((0, 0), (0, 1)))
    w1p = jnp.pad(W1, ((0, 1), (0, 0)))
    b1r = b1.reshape(1, -1)
    b2r = b2.reshape(1, -1)
    bcr = bc.reshape(1, -1)
    wct = Wc[:H]
    wcb = Wc[H:]

    h = pl.pallas_call(
        _mlp_in_kernel,
        grid=(NB,),
        in_specs=[
            pl.BlockSpec((RB, 16), lambda i: (i, 0)),
            _full2((16, H)), _full2((1, H)), _full2((H, H)), _full2((1, H)),
        ],
        out_specs=pl.BlockSpec((RB, H), lambda i: (i, 0)),
        out_shape=jax.ShapeDtypeStruct((N, H), f32),
        compiler_params=pltpu.CompilerParams(
            dimension_semantics=("arbitrary",)),
    )(x16, w1p, b1r, W2, b2r)

    for _ in range(3):
        h = pl.pallas_call(
            _edge_kernel,
            grid=(NB,),
            in_specs=[
                pl.BlockSpec(memory_space=pltpu.SMEM),
                pl.BlockSpec(memory_space=pltpu.SMEM),
                pl.BlockSpec((1, RB, 1), lambda i: (i, 0, 0)),
                pl.BlockSpec((1, RB, 1), lambda i: (i, 0, 0)),
                _full2((N, H)),
                _full2((H, H)),
                _full2((H, H)),
                _full2((1, H)),
            ],
            out_specs=pl.BlockSpec((RB, H), lambda i: (i, 0)),
            out_shape=jax.ShapeDtypeStruct((N, H), f32),
            scratch_shapes=[pltpu.VMEM((NB, RB, RB), f32),
                            pltpu.VMEM((K, RB, 1), f32),
                            pltpu.VMEM((K * RB, H), f32)],
            compiler_params=pltpu.CompilerParams(
                dimension_semantics=("arbitrary",)),
        )(blo, bhi, rowlo3, rowhi3, h, wct, wcb, bcr)

    out = pl.pallas_call(
        _pool_kernel,
        grid=(1,),
        in_specs=[
            pl.BlockSpec((NB, 1, RB), lambda i: (0, 0, 0)),
            _full2((N, H)),
            _full2((H, 64)), _full2((1, 64)),
            _full2((64, 32)), _full2((1, 32)),
            _full2((32, 32)), _full2((1, 32)),
            _full2((32, 8)), _full2((1, 8)),
        ],
        out_specs=pl.BlockSpec((G, 8), lambda i: (0, 0)),
        out_shape=jax.ShapeDtypeStruct((G, 8), f32),
        compiler_params=pltpu.CompilerParams(
            dimension_semantics=("arbitrary",)),
    )(batchc, h, Wo1, bo1.reshape(1, -1), Wo2, bo2.reshape(1, -1),
      Wo3, bo3.reshape(1, -1), Wo4, bo4.reshape(1, -1))

    return (out, batch_pf)
